# baseline (speedup 1.0000x reference)
import jax
import jax.numpy as jnp
from jax.experimental import pallas as pl

B, F, C = 4096, 100, 100
BB = 128


def _onehot_body(idx_ref, out_ref):
    idx = idx_ref[...]
    classes = jax.lax.broadcasted_iota(jnp.int32, (BB, F, C), 2)
    out_ref[...] = (idx[:, :, None] == classes).astype(jnp.int32)


def kernel(tensor):
    idx2 = tensor.reshape(B, F)
    return pl.pallas_call(
        _onehot_body,
        grid=(B // BB,),
        in_specs=[pl.BlockSpec((BB, F), lambda i: (i, 0))],
        out_specs=pl.BlockSpec((BB, F, C), lambda i: (i, 0, 0)),
        out_shape=jax.ShapeDtypeStruct((B, F, C), jnp.int32),
    )(idx2)
